# baseline (speedup 1.0000x reference)
import dataclasses
import functools

import jax
import jax.numpy as jnp
from jax import lax
from jax.experimental import pallas as pl
from jax.experimental.pallas import tpu as pltpu
from jax.experimental.pallas import tpu_sc as plsc

N = 10000
NP = 10240
E = 320000
D = 128

NUM_WORKERS = 32
NODES_PER_W = NP // NUM_WORKERS
EDGES_PER_W = E // NUM_WORKERS
CHUNK = 2000
SUB = 80
N_SUB = CHUNK // SUB
N_CHUNK = EDGES_PER_W // CHUNK
NBUF = 4
GW = 80

_vec_mesh = plsc.VectorSubcoreMesh(core_axis_name="c", subcore_axis_name="s")

_sc_params = pltpu.CompilerParams()
if "needs_layout_passes" in pltpu.CompilerParams.__dataclass_fields__:
    _sc_params = dataclasses.replace(_sc_params, needs_layout_passes=False)


@functools.partial(jax.jit, static_argnums=())
def _sc_gather_hist(emb_table, nodes_pad, col3):
    @functools.partial(
        pl.kernel,
        out_type=[
            jax.ShapeDtypeStruct((NP, D), jnp.float32),
            jax.ShapeDtypeStruct((NUM_WORKERS * NP,), jnp.float32),
        ],
        mesh=_vec_mesh,
        scratch_types=[
            pltpu.VMEM((NODES_PER_W,), jnp.int32),
            pltpu.VMEM((NODES_PER_W, D), jnp.float32),
            pltpu.VMEM((N_SUB, SUB), jnp.int32),
            pltpu.VMEM((N_SUB, SUB), jnp.int32),
            pltpu.VMEM((N_SUB, SUB), jnp.int32),
            pltpu.VMEM((N_SUB, SUB), jnp.int32),
            pltpu.VMEM((N_SUB, SUB), jnp.int32),
            pltpu.VMEM((NP,), jnp.float32),
            pltpu.SemaphoreType.DMA,
            pltpu.SemaphoreType.DMA,
        ],
        compiler_params=_sc_params,
    )
    def k(emb_hbm, nodes_hbm, col3_hbm, x0_hbm, deg_hbm, idx_v, rows_v,
          ch0, ch1, ch2, ch3, ch4, deg_l, sem, hsem):
        colh = [ch0, ch1, ch2, ch3, ch4]
        wid = lax.axis_index("c") * 16 + lax.axis_index("s")
        nbase = wid * NODES_PER_W

        pltpu.sync_copy(nodes_hbm.at[pl.ds(nbase, NODES_PER_W)], idx_v)
        copies = []
        for j in range(NODES_PER_W // GW):
            copies.append(pltpu.async_copy(
                emb_hbm.at[idx_v.at[pl.ds(j * GW, GW)]],
                rows_v.at[pl.ds(j * GW, GW)],
                sem,
            ))
        hcopies = []
        for q in range(N_CHUNK):
            hcopies.append(pltpu.async_copy(
                col3_hbm.at[wid * N_CHUNK + q], colh[q], hsem))

        zero16 = jnp.zeros((16,), jnp.float32)
        one16 = jnp.ones((16,), jnp.float32)

        @pl.loop(0, NP, step=16)
        def _(i):
            deg_l[pl.ds(i, 16)] = zero16

        for q in range(N_CHUNK):
            hcopies[q].wait()
            for r in range(N_SUB):
                for cc in range(SUB // 16):
                    plsc.addupdate_scatter(
                        deg_l, [colh[q][r, pl.ds(cc * 16, 16)]], one16)

        pltpu.sync_copy(deg_l, deg_hbm.at[pl.ds(wid * NP, NP)])

        for c in copies:
            c.wait()
        pltpu.sync_copy(rows_v, x0_hbm.at[pl.ds(nbase, NODES_PER_W)])

    return k(emb_table, nodes_pad, col3)


ROWS_BLK = 1024
N_BLK = NP // ROWS_BLK


def _tc_transform_body(x0_ref, w_ref, deg_ref, y_ref, dis_ref):
    ones = jnp.ones((NUM_WORKERS, 1), jnp.float32)
    s = lax.dot_general(deg_ref[...], ones, (((0,), (0,)), ((), ())),
                        preferred_element_type=jnp.float32)
    dis = lax.rsqrt(s + 1.0)
    xw = jnp.dot(x0_ref[...], w_ref[...],
                 preferred_element_type=jnp.float32,
                 precision=lax.Precision.DEFAULT)
    y_ref[...] = xw * dis
    dis_ref[...] = dis


@jax.jit
def _tc_transform(x0, W, deg_t):
    return pl.pallas_call(
        _tc_transform_body,
        grid=(N_BLK,),
        in_specs=[
            pl.BlockSpec((ROWS_BLK, D), lambda i: (i, 0)),
            pl.BlockSpec((D, D), lambda i: (0, 0)),
            pl.BlockSpec((NUM_WORKERS, ROWS_BLK), lambda i: (0, i)),
        ],
        out_specs=[
            pl.BlockSpec((ROWS_BLK, D), lambda i: (i, 0)),
            pl.BlockSpec((ROWS_BLK, 1), lambda i: (i, 0)),
        ],
        out_shape=[
            jax.ShapeDtypeStruct((NP, D), jnp.float32),
            jax.ShapeDtypeStruct((NP, 1), jnp.float32),
        ],
    )(x0, W, deg_t)


@jax.jit
def _sc_scatter(y, row3, col3, zeros_init):
    @functools.partial(
        pl.kernel,
        out_type=jax.ShapeDtypeStruct((2, NP, D), jnp.float32),
        mesh=_vec_mesh,
        scratch_types=[
            pltpu.VMEM((N_SUB, SUB), jnp.int32),
            pltpu.VMEM((N_SUB, SUB), jnp.int32),
            pltpu.VMEM((NBUF, SUB, D), jnp.float32),
            pltpu.VMEM_SHARED((NP, D), jnp.float32),
            pltpu.SemaphoreType.DMA,
            pltpu.SemaphoreType.DMA,
        ],
    )
    def k(y_hbm, row3_hbm, col3_hbm, zero_hbm, acc_hbm, row_v, col_v, buf,
          acc_s, gsem, ssem):
        c = lax.axis_index("c")
        s = lax.axis_index("s")
        wid = c * 16 + s
        slice_rows = NP // 16

        pltpu.sync_copy(zero_hbm, acc_s.at[pl.ds(s * slice_rows, slice_rows)])
        plsc.subcore_barrier()

        def gather(j, b):
            return pltpu.async_copy(
                y_hbm.at[row_v.at[j]], buf.at[b], gsem)

        def scatter(j, b):
            return pltpu.async_copy(buf.at[b], acc_s.at[col_v.at[j]], ssem,
                                    add=True)

        @pl.loop(0, N_CHUNK)
        def _(kk):
            ck = wid * N_CHUNK + kk
            pltpu.sync_copy(row3_hbm.at[ck], row_v)
            pltpu.sync_copy(col3_hbm.at[ck], col_v)
            g = [gather(j, j) for j in range(NBUF - 1)]
            sc = []
            waited = 0
            for j in range(N_SUB):
                g[j].wait()
                sc.append(scatter(j, j % NBUF))
                if j + NBUF - 1 < N_SUB:
                    if j >= 1:
                        sc[j - 1].wait()
                        waited = j
                    g.append(gather(j + NBUF - 1, (j + NBUF - 1) % NBUF))
            for j in range(waited, N_SUB):
                sc[j].wait()

        plsc.subcore_barrier()
        pltpu.sync_copy(
            acc_s.at[pl.ds(s * slice_rows, slice_rows)],
            acc_hbm.at[c, pl.ds(s * slice_rows, slice_rows)],
        )

    return k(y, row3, col3, zeros_init)


def _tc_combine_body(accs_ref, y_ref, dis_ref, b_ref, out_ref):
    acc = accs_ref[0] + accs_ref[1] + y_ref[...]
    out_ref[...] = dis_ref[...] * acc + b_ref[...]


OUT_BLK = 2000


@jax.jit
def _tc_combine(accs, y, dis, b2d):
    return pl.pallas_call(
        _tc_combine_body,
        grid=(N // OUT_BLK,),
        in_specs=[
            pl.BlockSpec((2, OUT_BLK, D), lambda i: (0, i, 0)),
            pl.BlockSpec((OUT_BLK, D), lambda i: (i, 0)),
            pl.BlockSpec((OUT_BLK, 1), lambda i: (i, 0)),
            pl.BlockSpec((1, D), lambda i: (0, 0)),
        ],
        out_specs=pl.BlockSpec((OUT_BLK, D), lambda i: (i, 0)),
        out_shape=jax.ShapeDtypeStruct((N, D), jnp.float32),
    )(accs, y, dis, b2d)


def kernel(nodes, edges, emb_table, W, b):
    nodes = nodes.astype(jnp.int32)
    edges = edges.astype(jnp.int32)
    col3 = edges[1].reshape(E // CHUNK, N_SUB, SUB)
    row3 = edges[0].reshape(E // CHUNK, N_SUB, SUB)
    nodes_pad = jnp.concatenate(
        [nodes, jnp.zeros((NP - N,), jnp.int32)])
    zeros_init = jnp.zeros((NP // 16, D), jnp.float32)

    x0, deg_flat = _sc_gather_hist(emb_table, nodes_pad, col3)
    y, dis = _tc_transform(x0, W, deg_flat.reshape(NUM_WORKERS, NP))
    accs = _sc_scatter(y, row3, col3, zeros_init)
    return _tc_combine(accs, y, dis, b.reshape(1, D))


# speedup vs baseline: 1.0576x; 1.0576x over previous
import dataclasses
import functools

import jax
import jax.numpy as jnp
from jax import lax
from jax.experimental import pallas as pl
from jax.experimental.pallas import tpu as pltpu
from jax.experimental.pallas import tpu_sc as plsc

N = 10000
NP = 10240
E = 320000
D = 128

NUM_WORKERS = 32
NODES_PER_W = NP // NUM_WORKERS
EDGES_PER_W = E // NUM_WORKERS
CHUNK = 2000
SUB = 80
N_SUB = CHUNK // SUB
N_CHUNK = EDGES_PER_W // CHUNK
NBUF = 4
GW = 80

_vec_mesh = plsc.VectorSubcoreMesh(core_axis_name="c", subcore_axis_name="s")

_sc_params = pltpu.CompilerParams()
if "needs_layout_passes" in pltpu.CompilerParams.__dataclass_fields__:
    _sc_params = dataclasses.replace(_sc_params, needs_layout_passes=False)


@functools.partial(jax.jit, static_argnums=())
def _sc_gather_hist(emb_table, nodes_pad, edges4):
    @functools.partial(
        pl.kernel,
        out_type=[
            jax.ShapeDtypeStruct((NP, D), jnp.float32),
            jax.ShapeDtypeStruct((NUM_WORKERS * NP,), jnp.float32),
        ],
        mesh=_vec_mesh,
        scratch_types=[
            pltpu.VMEM((NODES_PER_W,), jnp.int32),
            pltpu.VMEM((NODES_PER_W, D), jnp.float32),
            pltpu.VMEM((N_SUB, SUB), jnp.int32),
            pltpu.VMEM((N_SUB, SUB), jnp.int32),
            pltpu.VMEM((N_SUB, SUB), jnp.int32),
            pltpu.VMEM((N_SUB, SUB), jnp.int32),
            pltpu.VMEM((N_SUB, SUB), jnp.int32),
            pltpu.VMEM((NP,), jnp.float32),
            pltpu.SemaphoreType.DMA,
            pltpu.SemaphoreType.DMA,
        ],
        compiler_params=_sc_params,
    )
    def k(emb_hbm, nodes_hbm, e4_hbm, x0_hbm, deg_hbm, idx_v, rows_v,
          ch0, ch1, ch2, ch3, ch4, deg_l, sem, hsem):
        colh = [ch0, ch1, ch2, ch3, ch4]
        wid = lax.axis_index("c") * 16 + lax.axis_index("s")
        nbase = wid * NODES_PER_W

        pltpu.sync_copy(nodes_hbm.at[pl.ds(nbase, NODES_PER_W)], idx_v)
        copies = []
        for j in range(NODES_PER_W // GW):
            copies.append(pltpu.async_copy(
                emb_hbm.at[idx_v.at[pl.ds(j * GW, GW)]],
                rows_v.at[pl.ds(j * GW, GW)],
                sem,
            ))
        hcopies = []
        for q in range(N_CHUNK):
            hcopies.append(pltpu.async_copy(
                e4_hbm.at[1, wid * N_CHUNK + q], colh[q], hsem))

        zero16 = jnp.zeros((16,), jnp.float32)
        one16 = jnp.ones((16,), jnp.float32)

        @pl.loop(0, NP, step=16)
        def _(i):
            deg_l[pl.ds(i, 16)] = zero16

        for q in range(N_CHUNK):
            hcopies[q].wait()
            for r in range(N_SUB):
                for cc in range(SUB // 16):
                    plsc.addupdate_scatter(
                        deg_l, [colh[q][r, pl.ds(cc * 16, 16)]], one16)

        pltpu.sync_copy(deg_l, deg_hbm.at[pl.ds(wid * NP, NP)])

        for c in copies:
            c.wait()
        pltpu.sync_copy(rows_v, x0_hbm.at[pl.ds(nbase, NODES_PER_W)])

    return k(emb_table, nodes_pad, edges4)


ROWS_BLK = 1024
N_BLK = NP // ROWS_BLK


def _tc_transform_body(x0_ref, w_ref, deg_ref, y_ref, dis_ref):
    ones = jnp.ones((NUM_WORKERS, 1), jnp.float32)
    s = lax.dot_general(deg_ref[...], ones, (((0,), (0,)), ((), ())),
                        preferred_element_type=jnp.float32)
    dis = lax.rsqrt(s + 1.0)
    xw = jnp.dot(x0_ref[...], w_ref[...],
                 preferred_element_type=jnp.float32,
                 precision=lax.Precision.DEFAULT)
    y_ref[...] = xw * dis
    dis_ref[...] = dis


@jax.jit
def _tc_transform(x0, W, deg_t):
    return pl.pallas_call(
        _tc_transform_body,
        grid=(N_BLK,),
        in_specs=[
            pl.BlockSpec((ROWS_BLK, D), lambda i: (i, 0)),
            pl.BlockSpec((D, D), lambda i: (0, 0)),
            pl.BlockSpec((NUM_WORKERS, ROWS_BLK), lambda i: (0, i)),
        ],
        out_specs=[
            pl.BlockSpec((ROWS_BLK, D), lambda i: (i, 0)),
            pl.BlockSpec((ROWS_BLK, 1), lambda i: (i, 0)),
        ],
        out_shape=[
            jax.ShapeDtypeStruct((NP, D), jnp.float32),
            jax.ShapeDtypeStruct((NP, 1), jnp.float32),
        ],
    )(x0, W, deg_t)


@jax.jit
def _sc_scatter(y, edges4, zeros_init):
    @functools.partial(
        pl.kernel,
        out_type=jax.ShapeDtypeStruct((2, NP, D), jnp.float32),
        mesh=_vec_mesh,
        scratch_types=[
            pltpu.VMEM((N_SUB, SUB), jnp.int32),
            pltpu.VMEM((N_SUB, SUB), jnp.int32),
            pltpu.VMEM((NBUF, SUB, D), jnp.float32),
            pltpu.VMEM_SHARED((NP, D), jnp.float32),
            pltpu.SemaphoreType.DMA,
            pltpu.SemaphoreType.DMA,
        ],
    )
    def k(y_hbm, e4_hbm, zero_hbm, acc_hbm, row_v, col_v, buf,
          acc_s, gsem, ssem):
        c = lax.axis_index("c")
        s = lax.axis_index("s")
        wid = c * 16 + s
        slice_rows = NP // 16

        pltpu.sync_copy(zero_hbm, acc_s.at[pl.ds(s * slice_rows, slice_rows)])
        plsc.subcore_barrier()

        def gather(j, b):
            return pltpu.async_copy(
                y_hbm.at[row_v.at[j]], buf.at[b], gsem)

        def scatter(j, b):
            return pltpu.async_copy(buf.at[b], acc_s.at[col_v.at[j]], ssem,
                                    add=True)

        @pl.loop(0, N_CHUNK)
        def _(kk):
            ck = wid * N_CHUNK + kk
            pltpu.sync_copy(e4_hbm.at[0, ck], row_v)
            pltpu.sync_copy(e4_hbm.at[1, ck], col_v)
            g = [gather(j, j) for j in range(NBUF - 1)]
            sc = []
            waited = 0
            for j in range(N_SUB):
                g[j].wait()
                sc.append(scatter(j, j % NBUF))
                if j + NBUF - 1 < N_SUB:
                    if j >= 1:
                        sc[j - 1].wait()
                        waited = j
                    g.append(gather(j + NBUF - 1, (j + NBUF - 1) % NBUF))
            for j in range(waited, N_SUB):
                sc[j].wait()

        plsc.subcore_barrier()
        pltpu.sync_copy(
            acc_s.at[pl.ds(s * slice_rows, slice_rows)],
            acc_hbm.at[c, pl.ds(s * slice_rows, slice_rows)],
        )

    return k(y, edges4, zeros_init)


def _tc_combine_body(accs_ref, y_ref, dis_ref, b_ref, out_ref):
    acc = accs_ref[0] + accs_ref[1] + y_ref[...]
    out_ref[...] = dis_ref[...] * acc + b_ref[...]


OUT_BLK = 2000


@jax.jit
def _tc_combine(accs, y, dis, b2d):
    return pl.pallas_call(
        _tc_combine_body,
        grid=(N // OUT_BLK,),
        in_specs=[
            pl.BlockSpec((2, OUT_BLK, D), lambda i: (0, i, 0)),
            pl.BlockSpec((OUT_BLK, D), lambda i: (i, 0)),
            pl.BlockSpec((OUT_BLK, 1), lambda i: (i, 0)),
            pl.BlockSpec((1, D), lambda i: (0, 0)),
        ],
        out_specs=pl.BlockSpec((OUT_BLK, D), lambda i: (i, 0)),
        out_shape=jax.ShapeDtypeStruct((N, D), jnp.float32),
    )(accs, y, dis, b2d)


def kernel(nodes, edges, emb_table, W, b):
    nodes = nodes.astype(jnp.int32)
    edges4 = edges.astype(jnp.int32).reshape(2, E // CHUNK, N_SUB, SUB)
    nodes_pad = jnp.concatenate(
        [nodes, jnp.zeros((NP - N,), jnp.int32)])
    zeros_init = jnp.zeros((NP // 16, D), jnp.float32)

    x0, deg_flat = _sc_gather_hist(emb_table, nodes_pad, edges4)
    y, dis = _tc_transform(x0, W, deg_flat.reshape(NUM_WORKERS, NP))
    accs = _sc_scatter(y, edges4, zeros_init)
    return _tc_combine(accs, y, dis, b.reshape(1, D))
